# baseline (speedup 1.0000x reference)
import functools

import jax
import jax.numpy as jnp
from jax import lax
from jax.experimental import pallas as pl
from jax.experimental.pallas import tpu as pltpu
from jax.experimental.pallas import tpu_sc as plsc

USER_COUNT = 1000000
ITEM_COUNT = 1000000
DIM = 32
BATCH = 16384

NUM_CORES = 2
NUM_SUBCORES = 16
NUM_WORKERS = NUM_CORES * NUM_SUBCORES
BPW = BATCH // NUM_WORKERS
L = 16
WAVE = 8
KC = 128
NWAVE = BPW // WAVE


def _g_body(u_hbm, p_hbm, n_hbm, ut_hbm, it_hbm,
            ou, oi, oj,
            iu, ip, inn, su, sp, sn, bu, bp, bn, semu, semp, semn):
    wid = lax.axis_index("s") * NUM_CORES + lax.axis_index("c")
    base = wid * BPW
    lane_iota = lax.iota(jnp.int32, L)

    ut4 = ut_hbm.reshape(4, 8, USER_COUNT)
    it4 = it_hbm.reshape(4, 8, ITEM_COUNT)
    streams = (
        (iu, ut4, su, bu, ou, semu),
        (ip, it4, sp, bp, oi, semp),
        (inn, it4, sn, bn, oj, semn),
    )

    pltpu.sync_copy(u_hbm.at[pl.ds(base, BPW)], iu)
    pltpu.sync_copy(p_hbm.at[pl.ds(base, BPW)], ip)
    pltpu.sync_copy(n_hbm.at[pl.ds(base, BPW)], inn)

    def row_scalar(idxv, k):
        gb = pl.multiple_of((k >> 4) << 4, L)
        grp = idxv[pl.ds(gb, L)]
        return jnp.sum(jnp.where(lane_iota == (k & (L - 1)), grp, 0))

    def enqueue(idxv, table, stg, sem, w):
        for j in range(WAVE):
            r = row_scalar(idxv, w * WAVE + j)
            tc = pl.multiple_of((r >> 7) << 7, 128)
            for b in range(4):
                pltpu.async_copy(table.at[b, :, pl.ds(tc, 128)],
                                 stg.at[j].at[pl.ds(b * 8, 8)], sem)

    for idxv, table, stg, _, _, sem in streams:
        enqueue(idxv, table, stg, sem, 0)

    def wave(w, carry):
        for idxv, table, stg, obuf, out, sem in streams:
            for j in range(WAVE):
                pltpu.make_async_copy(
                    table.at[0, :, pl.ds(0, 128)], stg.at[j], sem).wait()
            for j in range(WAVE):
                r = row_scalar(idxv, w * WAVE + j)
                lvec = jnp.broadcast_to(r & 127, (L,))
                k = (w * WAVE + j) & (KC - 1)
                kvec = jnp.full((L,), k, jnp.int32)
                lo = plsc.load_gather(stg.at[j], [lane_iota, lvec])
                hi = plsc.load_gather(stg.at[j], [lane_iota + L, lvec])
                plsc.store_scatter(obuf, [lane_iota, kvec], lo)
                plsc.store_scatter(obuf, [lane_iota + L, kvec], hi)

            @pl.when(w < NWAVE - 1)
            def _():
                enqueue(idxv, table, stg, sem, w + 1)

            @pl.when(lax.rem(w, KC // WAVE) == KC // WAVE - 1)
            def _():
                cb = (w // (KC // WAVE)) * KC
                pltpu.sync_copy(
                    obuf,
                    out.at[:, pl.ds(pl.multiple_of(base + cb, 128), KC)])
        return carry

    lax.fori_loop(0, NWAVE, wave, 0)


@jax.jit
def kernel(u, p, n, user_emb, item_emb):
    u = jnp.asarray(u, jnp.int32)
    p = jnp.asarray(p, jnp.int32)
    n = jnp.asarray(n, jnp.int32)
    ut = user_emb.T
    it = item_emb.T
    mesh = plsc.VectorSubcoreMesh(
        core_axis_name="c", subcore_axis_name="s",
        num_cores=NUM_CORES, num_subcores=NUM_SUBCORES)
    out = jax.ShapeDtypeStruct((DIM, BATCH), jnp.float32)
    idx_t = pltpu.VMEM((BPW,), jnp.int32)
    stg_t = pltpu.VMEM((WAVE, DIM, 128), jnp.float32)
    obuf_t = pltpu.VMEM((DIM, KC), jnp.float32)
    run = pl.kernel(
        _g_body,
        out_type=(out, out, out),
        mesh=mesh,
        scratch_types=[
            idx_t, idx_t, idx_t,
            stg_t, stg_t, stg_t,
            obuf_t, obuf_t, obuf_t,
            pltpu.SemaphoreType.DMA,
            pltpu.SemaphoreType.DMA,
            pltpu.SemaphoreType.DMA,
        ],
        compiler_params=pltpu.CompilerParams(needs_layout_passes=False),
    )
    ou, oi, oj = run(u, p, n, ut, it)
    return (ou.T, oi.T, oj.T)
